# baseline (speedup 1.0000x reference)
import jax
import jax.numpy as jnp
from jax import lax
from jax.experimental import pallas as pl
from jax.experimental.pallas import tpu as pltpu
from jax.experimental.pallas import tpu_sc as plsc

N = 10000
E = 160000
D_IN = 300
D_HID = 150
DA = 128
DB = 32
DP = DA + DB
DEGC = D_HID - DA

NC, NS = 2, 16
NW = NC * NS
CHUNK = 125
NCHUNK = E // (NW * CHUNK)
NPAD = 10240
NROWS_PER_TILE = NPAD // NS

BM = 2000
GRID = N // BM


_NT = (((1,), (1,)), ((), ()))


def _split_ab(d):
    ones = jnp.ones((d.shape[0], 1), jnp.float32)
    zeros = jnp.zeros((d.shape[0], DB - DEGC - 1), jnp.float32)
    return d[:, :DA], jnp.concatenate([d[:, DA:], ones, zeros], axis=1)


def _k1_body(x_ref, w_ref, oa_ref, ob_ref):
    d = lax.dot_general(x_ref[...], w_ref[...], _NT,
                        preferred_element_type=jnp.float32)
    oa_ref[...], ob_ref[...] = _split_ab(d)


def _k3_body(sa_ref, sb_ref, x_ref, w_ref, b_ref, oa_ref, ob_ref):
    sa = sa_ref[0] + sa_ref[1]
    sb = sb_ref[0] + sb_ref[1]
    deg = jnp.maximum(sb[:, DEGC:DEGC + 1], 1.0)
    m = jnp.concatenate([sa, sb[:, :DEGC]], axis=1) / deg
    z = m + lax.dot_general(x_ref[...], w_ref[...], _NT,
                            preferred_element_type=jnp.float32) + b_ref[...]
    oa_ref[...], ob_ref[...] = _split_ab(jnp.maximum(z, 0.0))


def _k5_body(sa_ref, sb_ref, ha_ref, hb_ref, wl_ref, wr_ref, b_ref, o_ref):
    sa = sa_ref[0] + sa_ref[1]
    sb = sb_ref[0] + sb_ref[1]
    deg = jnp.maximum(sb[:, DEGC:DEGC + 1], 1.0)
    m = jnp.concatenate([sa, sb[:, :DEGC]], axis=1) / deg
    h = jnp.concatenate([ha_ref[...], hb_ref[:, :DEGC]], axis=1)
    o_ref[...] = (lax.dot_general(m, wl_ref[...], _NT,
                                  preferred_element_type=jnp.float32)
                  + lax.dot_general(h, wr_ref[...], _NT,
                                    preferred_element_type=jnp.float32)
                  + b_ref[...])


def _segsum_body(y_hbm, src_hbm, dst_hbm, zero_hbm, out_hbm,
                 src_v, dst_v, rows0_v, rows1_v, acc_sh,
                 gsem0, gsem1, ssem0, ssem1):
    cid = lax.axis_index("c")
    sid = lax.axis_index("s")
    wid = cid * NS + sid
    sl = pl.ds(sid * NROWS_PER_TILE, NROWS_PER_TILE)
    pltpu.sync_copy(zero_hbm.at[sl], acc_sh.at[sl])
    pltpu.sync_copy(src_hbm.at[wid], src_v)
    pltpu.sync_copy(dst_hbm.at[wid], dst_v)
    plsc.subcore_barrier()

    pltpu.async_copy(y_hbm.at[src_v.at[0]], rows0_v, gsem0)

    def chunk(j, carry):
        c = 2 * j
        pltpu.make_async_copy(y_hbm.at[src_v.at[c]], rows0_v, gsem0).wait()
        pltpu.async_copy(rows0_v, acc_sh.at[dst_v.at[c]], ssem0, add=True)

        @pl.when(j > 0)
        def _():
            pltpu.make_async_copy(rows1_v, acc_sh.at[dst_v.at[c - 1]],
                                  ssem1).wait()

        pltpu.async_copy(y_hbm.at[src_v.at[c + 1]], rows1_v, gsem1)
        pltpu.make_async_copy(y_hbm.at[src_v.at[c + 1]], rows1_v, gsem1).wait()
        pltpu.async_copy(rows1_v, acc_sh.at[dst_v.at[c + 1]], ssem1, add=True)
        pltpu.make_async_copy(rows0_v, acc_sh.at[dst_v.at[c]], ssem0).wait()

        @pl.when(j < NCHUNK // 2 - 1)
        def _():
            pltpu.async_copy(y_hbm.at[src_v.at[c + 2]], rows0_v, gsem0)

        return carry

    lax.fori_loop(0, NCHUNK // 2, chunk, 0)
    pltpu.make_async_copy(rows1_v, acc_sh.at[dst_v.at[NCHUNK - 1]],
                          ssem1).wait()
    plsc.subcore_barrier()
    pltpu.sync_copy(acc_sh.at[sl], out_hbm.at[cid, sl])


def _make_segsum(width, tc_tiling):
    return pl.kernel(
        _segsum_body,
        out_type=jax.ShapeDtypeStruct((NC, NPAD, width), jnp.float32),
        mesh=plsc.VectorSubcoreMesh(core_axis_name="c", subcore_axis_name="s"),
        scratch_types=[
            pltpu.VMEM((NCHUNK, CHUNK), jnp.int32),
            pltpu.VMEM((NCHUNK, CHUNK), jnp.int32),
            pltpu.VMEM((CHUNK, width), jnp.float32),
            pltpu.VMEM((CHUNK, width), jnp.float32),
            pltpu.VMEM_SHARED((NPAD, width), jnp.float32),
            pltpu.SemaphoreType.DMA,
            pltpu.SemaphoreType.DMA,
            pltpu.SemaphoreType.DMA,
            pltpu.SemaphoreType.DMA,
        ],
        compiler_params=pltpu.CompilerParams(use_tc_tiling_on_sc=tc_tiling),
    )


_segsum_a = _make_segsum(DA, True)
_segsum_b = _make_segsum(DB, False)


@jax.jit
def kernel(features, edges, W1_l, b1, W1_r, W2_l, b2, W2_r):
    f32 = jnp.float32
    b1p = b1[None, :]
    b2p = b2[None, :]
    src_r = edges[0].reshape(NW, NCHUNK, CHUNK)
    dst_r = edges[1].reshape(NW, NCHUNK, CHUNK)
    zeros_a = jnp.zeros((NPAD, DA), f32)
    zeros_b = jnp.zeros((NPAD, DB), f32)

    y1a, y1b = pl.pallas_call(
        _k1_body,
        grid=(GRID,),
        in_specs=[pl.BlockSpec((BM, D_IN), lambda i: (i, 0)),
                  pl.BlockSpec((D_HID, D_IN), lambda i: (0, 0))],
        out_specs=[pl.BlockSpec((BM, DA), lambda i: (i, 0)),
                   pl.BlockSpec((BM, DB), lambda i: (i, 0))],
        out_shape=[jax.ShapeDtypeStruct((N, DA), f32),
                   jax.ShapeDtypeStruct((N, DB), f32)],
    )(features, W1_l)

    seg1a = _segsum_a(y1a, src_r, dst_r, zeros_a)
    seg1b = _segsum_b(y1b, src_r, dst_r, zeros_b)

    h2a, h2b = pl.pallas_call(
        _k3_body,
        grid=(GRID,),
        in_specs=[pl.BlockSpec((NC, BM, DA), lambda i: (0, i, 0)),
                  pl.BlockSpec((NC, BM, DB), lambda i: (0, i, 0)),
                  pl.BlockSpec((BM, D_IN), lambda i: (i, 0)),
                  pl.BlockSpec((D_HID, D_IN), lambda i: (0, 0)),
                  pl.BlockSpec((1, D_HID), lambda i: (0, 0))],
        out_specs=[pl.BlockSpec((BM, DA), lambda i: (i, 0)),
                   pl.BlockSpec((BM, DB), lambda i: (i, 0))],
        out_shape=[jax.ShapeDtypeStruct((N, DA), f32),
                   jax.ShapeDtypeStruct((N, DB), f32)],
    )(seg1a, seg1b, features, W1_r, b1p)

    seg2a = _segsum_a(h2a, src_r, dst_r, zeros_a)
    seg2b = _segsum_b(h2b, src_r, dst_r, zeros_b)

    out = pl.pallas_call(
        _k5_body,
        grid=(GRID,),
        in_specs=[pl.BlockSpec((NC, BM, DA), lambda i: (0, i, 0)),
                  pl.BlockSpec((NC, BM, DB), lambda i: (0, i, 0)),
                  pl.BlockSpec((BM, DA), lambda i: (i, 0)),
                  pl.BlockSpec((BM, DB), lambda i: (i, 0)),
                  pl.BlockSpec((D_IN, D_HID), lambda i: (0, 0)),
                  pl.BlockSpec((D_IN, D_HID), lambda i: (0, 0)),
                  pl.BlockSpec((1, D_IN), lambda i: (0, 0))],
        out_specs=pl.BlockSpec((BM, D_IN), lambda i: (i, 0)),
        out_shape=jax.ShapeDtypeStruct((N, D_IN), f32),
    )(seg2a, seg2b, h2a, h2b, W2_l, W2_r, b2p)
    return out


# speedup vs baseline: 1.1425x; 1.1425x over previous
import jax
import jax.numpy as jnp
from jax import lax
from jax.experimental import pallas as pl
from jax.experimental.pallas import tpu as pltpu
from jax.experimental.pallas import tpu_sc as plsc

N = 10000
E = 160000
D_IN = 300
D_HID = 150
DA = 128
DB = 32
DP = DA + DB
DEGC = D_HID - DA

NC, NS = 2, 16
NW = NC * NS
CHUNK = 125
NCHUNK = E // (NW * CHUNK)
NPAD = 10240
NROWS_PER_TILE = NPAD // NS

BM = 2000
GRID = N // BM


_NT = (((1,), (1,)), ((), ()))


def _split_ab(d):
    ones = jnp.ones((d.shape[0], 1), jnp.float32)
    zeros = jnp.zeros((d.shape[0], DB - DEGC - 1), jnp.float32)
    return d[:, :DA], jnp.concatenate([d[:, DA:], ones, zeros], axis=1)


def _k1_body(x_ref, w_ref, oa_ref, ob_ref):
    d = lax.dot_general(x_ref[...], w_ref[...], _NT,
                        preferred_element_type=jnp.float32)
    oa_ref[...], ob_ref[...] = _split_ab(d)


def _k3_body(sa_ref, sb_ref, x_ref, w_ref, b_ref, oa_ref, ob_ref):
    sa = sa_ref[0] + sa_ref[1]
    sb = sb_ref[0] + sb_ref[1]
    deg = jnp.maximum(sb[:, DEGC:DEGC + 1], 1.0)
    m = jnp.concatenate([sa, sb[:, :DEGC]], axis=1) / deg
    z = m + lax.dot_general(x_ref[...], w_ref[...], _NT,
                            preferred_element_type=jnp.float32) + b_ref[...]
    oa_ref[...], ob_ref[...] = _split_ab(jnp.maximum(z, 0.0))


def _k5_body(sa_ref, sb_ref, ha_ref, hb_ref, wl_ref, wr_ref, b_ref, o_ref):
    sa = sa_ref[0] + sa_ref[1]
    sb = sb_ref[0] + sb_ref[1]
    deg = jnp.maximum(sb[:, DEGC:DEGC + 1], 1.0)
    m = jnp.concatenate([sa, sb[:, :DEGC]], axis=1) / deg
    h = jnp.concatenate([ha_ref[...], hb_ref[:, :DEGC]], axis=1)
    o_ref[...] = (lax.dot_general(m, wl_ref[...], _NT,
                                  preferred_element_type=jnp.float32)
                  + lax.dot_general(h, wr_ref[...], _NT,
                                    preferred_element_type=jnp.float32)
                  + b_ref[...])


def _segsum_body(y_hbm, src_hbm, dst_hbm, zero_hbm, out_hbm,
                 src_v, dst_v, rows0_v, rows1_v, acc_sh,
                 gsem0, gsem1, ssem0, ssem1):
    cid = lax.axis_index("c")
    sid = lax.axis_index("s")
    wid = cid * NS + sid
    sl = pl.ds(sid * NROWS_PER_TILE, NROWS_PER_TILE)
    pltpu.sync_copy(zero_hbm.at[sl], acc_sh.at[sl])
    pltpu.sync_copy(src_hbm.at[wid], src_v)
    pltpu.sync_copy(dst_hbm.at[wid], dst_v)
    plsc.subcore_barrier()

    pltpu.async_copy(y_hbm.at[src_v.at[0]], rows0_v, gsem0)

    def chunk(j, carry):
        c = 2 * j
        pltpu.async_copy(y_hbm.at[src_v.at[c + 1]], rows1_v, gsem1)
        pltpu.make_async_copy(y_hbm.at[src_v.at[c]], rows0_v, gsem0).wait()
        pltpu.sync_copy(rows0_v, acc_sh.at[dst_v.at[c]], add=True)

        @pl.when(j < NCHUNK // 2 - 1)
        def _():
            pltpu.async_copy(y_hbm.at[src_v.at[c + 2]], rows0_v, gsem0)

        pltpu.make_async_copy(y_hbm.at[src_v.at[c + 1]], rows1_v, gsem1).wait()
        pltpu.sync_copy(rows1_v, acc_sh.at[dst_v.at[c + 1]], add=True)
        return carry

    lax.fori_loop(0, NCHUNK // 2, chunk, 0)
    plsc.subcore_barrier()
    pltpu.sync_copy(acc_sh.at[sl], out_hbm.at[cid, sl])


def _make_segsum(width, tc_tiling):
    return pl.kernel(
        _segsum_body,
        out_type=jax.ShapeDtypeStruct((NC, NPAD, width), jnp.float32),
        mesh=plsc.VectorSubcoreMesh(core_axis_name="c", subcore_axis_name="s"),
        scratch_types=[
            pltpu.VMEM((NCHUNK, CHUNK), jnp.int32),
            pltpu.VMEM((NCHUNK, CHUNK), jnp.int32),
            pltpu.VMEM((CHUNK, width), jnp.float32),
            pltpu.VMEM((CHUNK, width), jnp.float32),
            pltpu.VMEM_SHARED((NPAD, width), jnp.float32),
            pltpu.SemaphoreType.DMA,
            pltpu.SemaphoreType.DMA,
            pltpu.SemaphoreType.DMA,
            pltpu.SemaphoreType.DMA,
        ],
        compiler_params=pltpu.CompilerParams(use_tc_tiling_on_sc=tc_tiling),
    )


_segsum_a = _make_segsum(DA, True)
_segsum_b = _make_segsum(DB, False)


@jax.jit
def kernel(features, edges, W1_l, b1, W1_r, W2_l, b2, W2_r):
    f32 = jnp.float32
    b1p = b1[None, :]
    b2p = b2[None, :]
    src_r = edges[0].reshape(NW, NCHUNK, CHUNK)
    dst_r = edges[1].reshape(NW, NCHUNK, CHUNK)
    zeros_a = jnp.zeros((NPAD, DA), f32)
    zeros_b = jnp.zeros((NPAD, DB), f32)

    y1a, y1b = pl.pallas_call(
        _k1_body,
        grid=(GRID,),
        in_specs=[pl.BlockSpec((BM, D_IN), lambda i: (i, 0)),
                  pl.BlockSpec((D_HID, D_IN), lambda i: (0, 0))],
        out_specs=[pl.BlockSpec((BM, DA), lambda i: (i, 0)),
                   pl.BlockSpec((BM, DB), lambda i: (i, 0))],
        out_shape=[jax.ShapeDtypeStruct((N, DA), f32),
                   jax.ShapeDtypeStruct((N, DB), f32)],
    )(features, W1_l)

    seg1a = _segsum_a(y1a, src_r, dst_r, zeros_a)
    seg1b = _segsum_b(y1b, src_r, dst_r, zeros_b)

    h2a, h2b = pl.pallas_call(
        _k3_body,
        grid=(GRID,),
        in_specs=[pl.BlockSpec((NC, BM, DA), lambda i: (0, i, 0)),
                  pl.BlockSpec((NC, BM, DB), lambda i: (0, i, 0)),
                  pl.BlockSpec((BM, D_IN), lambda i: (i, 0)),
                  pl.BlockSpec((D_HID, D_IN), lambda i: (0, 0)),
                  pl.BlockSpec((1, D_HID), lambda i: (0, 0))],
        out_specs=[pl.BlockSpec((BM, DA), lambda i: (i, 0)),
                   pl.BlockSpec((BM, DB), lambda i: (i, 0))],
        out_shape=[jax.ShapeDtypeStruct((N, DA), f32),
                   jax.ShapeDtypeStruct((N, DB), f32)],
    )(seg1a, seg1b, features, W1_r, b1p)

    seg2a = _segsum_a(h2a, src_r, dst_r, zeros_a)
    seg2b = _segsum_b(h2b, src_r, dst_r, zeros_b)

    out = pl.pallas_call(
        _k5_body,
        grid=(GRID,),
        in_specs=[pl.BlockSpec((NC, BM, DA), lambda i: (0, i, 0)),
                  pl.BlockSpec((NC, BM, DB), lambda i: (0, i, 0)),
                  pl.BlockSpec((BM, DA), lambda i: (i, 0)),
                  pl.BlockSpec((BM, DB), lambda i: (i, 0)),
                  pl.BlockSpec((D_IN, D_HID), lambda i: (0, 0)),
                  pl.BlockSpec((D_IN, D_HID), lambda i: (0, 0)),
                  pl.BlockSpec((1, D_IN), lambda i: (0, 0))],
        out_specs=pl.BlockSpec((BM, D_IN), lambda i: (i, 0)),
        out_shape=jax.ShapeDtypeStruct((N, D_IN), f32),
    )(seg2a, seg2b, h2a, h2b, W2_l, W2_r, b2p)
    return out


# speedup vs baseline: 1.1920x; 1.0434x over previous
import jax
import jax.numpy as jnp
from jax import lax
from jax.experimental import pallas as pl
from jax.experimental.pallas import tpu as pltpu
from jax.experimental.pallas import tpu_sc as plsc

N = 10000
E = 160000
D_IN = 300
D_HID = 150
DA = 128
DB = 32
DP = DA + DB
DEGC = D_HID - DA

NC, NS = 2, 16
NW = NC * NS
CHUNK = 125
NCHUNK = E // (NW * CHUNK)
NPAD = 10240
NROWS_PER_TILE = NPAD // NS

BM = 2000
GRID = N // BM


_NT = (((1,), (1,)), ((), ()))


def _split_ab(d):
    ones = jnp.ones((d.shape[0], 1), jnp.float32)
    zeros = jnp.zeros((d.shape[0], DB - DEGC - 1), jnp.float32)
    return d[:, :DA], jnp.concatenate([d[:, DA:], ones, zeros], axis=1)


def _k1_body(x_ref, w_ref, oa_ref, ob_ref):
    d = lax.dot_general(x_ref[...], w_ref[...], _NT,
                        preferred_element_type=jnp.float32)
    oa, ob_ref[...] = _split_ab(d)
    oa_ref[...] = oa.astype(jnp.bfloat16)


def _k3_body(sa_ref, sb_ref, x_ref, w_ref, b_ref, oa_ref, ob_ref):
    sa = sa_ref[0].astype(jnp.float32) + sa_ref[1].astype(jnp.float32)
    sb = sb_ref[0] + sb_ref[1]
    deg = jnp.maximum(sb[:, DEGC:DEGC + 1], 1.0)
    m = jnp.concatenate([sa, sb[:, :DEGC]], axis=1) / deg
    z = m + lax.dot_general(x_ref[...], w_ref[...], _NT,
                            preferred_element_type=jnp.float32) + b_ref[...]
    oa, ob_ref[...] = _split_ab(jnp.maximum(z, 0.0))
    oa_ref[...] = oa.astype(jnp.bfloat16)


def _k5_body(sa_ref, sb_ref, ha_ref, hb_ref, wl_ref, wr_ref, b_ref, o_ref):
    sa = sa_ref[0].astype(jnp.float32) + sa_ref[1].astype(jnp.float32)
    sb = sb_ref[0] + sb_ref[1]
    deg = jnp.maximum(sb[:, DEGC:DEGC + 1], 1.0)
    m = jnp.concatenate([sa, sb[:, :DEGC]], axis=1) / deg
    h = jnp.concatenate([ha_ref[...].astype(jnp.float32),
                         hb_ref[:, :DEGC]], axis=1)
    o_ref[...] = (lax.dot_general(m, wl_ref[...], _NT,
                                  preferred_element_type=jnp.float32)
                  + lax.dot_general(h, wr_ref[...], _NT,
                                    preferred_element_type=jnp.float32)
                  + b_ref[...])


def _segsum_body(y_hbm, src_hbm, dst_hbm, zero_hbm, out_hbm,
                 src_v, dst_v, rows0_v, rows1_v, acc_sh,
                 gsem0, gsem1, ssem0, ssem1):
    cid = lax.axis_index("c")
    sid = lax.axis_index("s")
    wid = cid * NS + sid
    sl = pl.ds(sid * NROWS_PER_TILE, NROWS_PER_TILE)
    pltpu.sync_copy(zero_hbm.at[sl], acc_sh.at[sl])
    pltpu.sync_copy(src_hbm.at[wid], src_v)
    pltpu.sync_copy(dst_hbm.at[wid], dst_v)
    plsc.subcore_barrier()

    pltpu.async_copy(y_hbm.at[src_v.at[0]], rows0_v, gsem0)

    def chunk(j, carry):
        c = 2 * j
        pltpu.async_copy(y_hbm.at[src_v.at[c + 1]], rows1_v, gsem1)
        pltpu.make_async_copy(y_hbm.at[src_v.at[c]], rows0_v, gsem0).wait()
        pltpu.sync_copy(rows0_v, acc_sh.at[dst_v.at[c]], add=True)

        @pl.when(j < NCHUNK // 2 - 1)
        def _():
            pltpu.async_copy(y_hbm.at[src_v.at[c + 2]], rows0_v, gsem0)

        pltpu.make_async_copy(y_hbm.at[src_v.at[c + 1]], rows1_v, gsem1).wait()
        pltpu.sync_copy(rows1_v, acc_sh.at[dst_v.at[c + 1]], add=True)
        return carry

    lax.fori_loop(0, NCHUNK // 2, chunk, 0)
    plsc.subcore_barrier()
    pltpu.sync_copy(acc_sh.at[sl], out_hbm.at[cid, sl])


def _make_segsum(width, dtype, tc_tiling):
    return pl.kernel(
        _segsum_body,
        out_type=jax.ShapeDtypeStruct((NC, NPAD, width), dtype),
        mesh=plsc.VectorSubcoreMesh(core_axis_name="c", subcore_axis_name="s"),
        scratch_types=[
            pltpu.VMEM((NCHUNK, CHUNK), jnp.int32),
            pltpu.VMEM((NCHUNK, CHUNK), jnp.int32),
            pltpu.VMEM((CHUNK, width), dtype),
            pltpu.VMEM((CHUNK, width), dtype),
            pltpu.VMEM_SHARED((NPAD, width), dtype),
            pltpu.SemaphoreType.DMA,
            pltpu.SemaphoreType.DMA,
            pltpu.SemaphoreType.DMA,
            pltpu.SemaphoreType.DMA,
        ],
        compiler_params=pltpu.CompilerParams(use_tc_tiling_on_sc=tc_tiling),
    )


_segsum_a = _make_segsum(DA, jnp.bfloat16, False)
_segsum_b = _make_segsum(DB, jnp.float32, False)


@jax.jit
def kernel(features, edges, W1_l, b1, W1_r, W2_l, b2, W2_r):
    f32 = jnp.float32
    b1p = b1[None, :]
    b2p = b2[None, :]
    src_r = edges[0].reshape(NW, NCHUNK, CHUNK)
    dst_r = edges[1].reshape(NW, NCHUNK, CHUNK)
    zeros_a = jnp.zeros((NPAD, DA), jnp.bfloat16)
    zeros_b = jnp.zeros((NPAD, DB), f32)

    y1a, y1b = pl.pallas_call(
        _k1_body,
        grid=(GRID,),
        in_specs=[pl.BlockSpec((BM, D_IN), lambda i: (i, 0)),
                  pl.BlockSpec((D_HID, D_IN), lambda i: (0, 0))],
        out_specs=[pl.BlockSpec((BM, DA), lambda i: (i, 0)),
                   pl.BlockSpec((BM, DB), lambda i: (i, 0))],
        out_shape=[jax.ShapeDtypeStruct((N, DA), jnp.bfloat16),
                   jax.ShapeDtypeStruct((N, DB), f32)],
    )(features, W1_l)

    seg1a = _segsum_a(y1a, src_r, dst_r, zeros_a)
    seg1b = _segsum_b(y1b, src_r, dst_r, zeros_b)

    h2a, h2b = pl.pallas_call(
        _k3_body,
        grid=(GRID,),
        in_specs=[pl.BlockSpec((NC, BM, DA), lambda i: (0, i, 0)),
                  pl.BlockSpec((NC, BM, DB), lambda i: (0, i, 0)),
                  pl.BlockSpec((BM, D_IN), lambda i: (i, 0)),
                  pl.BlockSpec((D_HID, D_IN), lambda i: (0, 0)),
                  pl.BlockSpec((1, D_HID), lambda i: (0, 0))],
        out_specs=[pl.BlockSpec((BM, DA), lambda i: (i, 0)),
                   pl.BlockSpec((BM, DB), lambda i: (i, 0))],
        out_shape=[jax.ShapeDtypeStruct((N, DA), jnp.bfloat16),
                   jax.ShapeDtypeStruct((N, DB), f32)],
    )(seg1a, seg1b, features, W1_r, b1p)

    seg2a = _segsum_a(h2a, src_r, dst_r, zeros_a)
    seg2b = _segsum_b(h2b, src_r, dst_r, zeros_b)

    out = pl.pallas_call(
        _k5_body,
        grid=(GRID,),
        in_specs=[pl.BlockSpec((NC, BM, DA), lambda i: (0, i, 0)),
                  pl.BlockSpec((NC, BM, DB), lambda i: (0, i, 0)),
                  pl.BlockSpec((BM, DA), lambda i: (i, 0)),
                  pl.BlockSpec((BM, DB), lambda i: (i, 0)),
                  pl.BlockSpec((D_IN, D_HID), lambda i: (0, 0)),
                  pl.BlockSpec((D_IN, D_HID), lambda i: (0, 0)),
                  pl.BlockSpec((1, D_IN), lambda i: (0, 0))],
        out_specs=pl.BlockSpec((BM, D_IN), lambda i: (i, 0)),
        out_shape=jax.ShapeDtypeStruct((N, D_IN), f32),
    )(seg2a, seg2b, h2a, h2b, W2_l, W2_r, b2p)
    return out


# speedup vs baseline: 1.3554x; 1.1371x over previous
import jax
import jax.numpy as jnp
from jax import lax
from jax.experimental import pallas as pl
from jax.experimental.pallas import tpu as pltpu
from jax.experimental.pallas import tpu_sc as plsc

N = 10000
E = 160000
D_IN = 300
D_HID = 150
DA = 128
DB = 32
DP = DA + DB
DEGC = D_HID - DA

NC, NS = 2, 16
NW = NC * NS
CHUNK = 200
NCHUNK = E // (NW * CHUNK)
NPAD = 10240
NROWS_PER_TILE = NPAD // NS

BM = 2000
GRID = N // BM


_NT = (((1,), (1,)), ((), ()))


def _split_ab(d):
    ones = jnp.ones((d.shape[0], 1), jnp.float32)
    zeros = jnp.zeros((d.shape[0], DB - DEGC - 1), jnp.float32)
    return d[:, :DA], jnp.concatenate([d[:, DA:], ones, zeros], axis=1)


def _k1_body(x_ref, w_ref, oa_ref, ob_ref):
    d = lax.dot_general(x_ref[...], w_ref[...], _NT,
                        preferred_element_type=jnp.float32)
    oa, ob = _split_ab(d)
    oa_ref[...] = oa.astype(jnp.bfloat16)
    ob_ref[...] = ob.astype(jnp.bfloat16)


def _k3_body(sa_ref, sb_ref, x_ref, w_ref, b_ref, oa_ref, ob_ref):
    sa = sa_ref[0].astype(jnp.float32) + sa_ref[1].astype(jnp.float32)
    sb = sb_ref[0].astype(jnp.float32) + sb_ref[1].astype(jnp.float32)
    deg = jnp.maximum(sb[:, DEGC:DEGC + 1], 1.0)
    m = jnp.concatenate([sa, sb[:, :DEGC]], axis=1) / deg
    z = m + lax.dot_general(x_ref[...], w_ref[...], _NT,
                            preferred_element_type=jnp.float32) + b_ref[...]
    oa, ob = _split_ab(jnp.maximum(z, 0.0))
    oa_ref[...] = oa.astype(jnp.bfloat16)
    ob_ref[...] = ob.astype(jnp.bfloat16)


def _k5_body(sa_ref, sb_ref, ha_ref, hb_ref, wl_ref, wr_ref, b_ref, o_ref):
    sa = sa_ref[0].astype(jnp.float32) + sa_ref[1].astype(jnp.float32)
    sb = sb_ref[0].astype(jnp.float32) + sb_ref[1].astype(jnp.float32)
    deg = jnp.maximum(sb[:, DEGC:DEGC + 1], 1.0)
    m = jnp.concatenate([sa, sb[:, :DEGC]], axis=1) / deg
    h = jnp.concatenate([ha_ref[...].astype(jnp.float32),
                         hb_ref[:, :DEGC].astype(jnp.float32)], axis=1)
    o_ref[...] = (lax.dot_general(m, wl_ref[...], _NT,
                                  preferred_element_type=jnp.float32)
                  + lax.dot_general(h, wr_ref[...], _NT,
                                    preferred_element_type=jnp.float32)
                  + b_ref[...])


def _segsum_body(y_hbm, src_hbm, dst_hbm, zero_hbm, out_hbm,
                 src_v, dst_v, rows0_v, rows1_v, acc_sh,
                 gsem0, gsem1, ssem0, ssem1):
    cid = lax.axis_index("c")
    sid = lax.axis_index("s")
    wid = cid * NS + sid
    sl = pl.ds(sid * NROWS_PER_TILE, NROWS_PER_TILE)
    pltpu.sync_copy(zero_hbm.at[sl], acc_sh.at[sl])
    pltpu.sync_copy(src_hbm.at[wid], src_v)
    pltpu.sync_copy(dst_hbm.at[wid], dst_v)
    plsc.subcore_barrier()

    pltpu.async_copy(y_hbm.at[src_v.at[0]], rows0_v, gsem0)

    def chunk(j, carry):
        c = 2 * j
        pltpu.async_copy(y_hbm.at[src_v.at[c + 1]], rows1_v, gsem1)
        pltpu.make_async_copy(y_hbm.at[src_v.at[c]], rows0_v, gsem0).wait()
        pltpu.sync_copy(rows0_v, acc_sh.at[dst_v.at[c]], add=True)

        @pl.when(j < NCHUNK // 2 - 1)
        def _():
            pltpu.async_copy(y_hbm.at[src_v.at[c + 2]], rows0_v, gsem0)

        pltpu.make_async_copy(y_hbm.at[src_v.at[c + 1]], rows1_v, gsem1).wait()
        pltpu.sync_copy(rows1_v, acc_sh.at[dst_v.at[c + 1]], add=True)
        return carry

    lax.fori_loop(0, NCHUNK // 2, chunk, 0)
    plsc.subcore_barrier()
    pltpu.sync_copy(acc_sh.at[sl], out_hbm.at[cid, sl])


def _make_segsum(width, dtype, tc_tiling):
    return pl.kernel(
        _segsum_body,
        out_type=jax.ShapeDtypeStruct((NC, NPAD, width), dtype),
        mesh=plsc.VectorSubcoreMesh(core_axis_name="c", subcore_axis_name="s"),
        scratch_types=[
            pltpu.VMEM((NCHUNK, CHUNK), jnp.int32),
            pltpu.VMEM((NCHUNK, CHUNK), jnp.int32),
            pltpu.VMEM((CHUNK, width), dtype),
            pltpu.VMEM((CHUNK, width), dtype),
            pltpu.VMEM_SHARED((NPAD, width), dtype),
            pltpu.SemaphoreType.DMA,
            pltpu.SemaphoreType.DMA,
            pltpu.SemaphoreType.DMA,
            pltpu.SemaphoreType.DMA,
        ],
        compiler_params=pltpu.CompilerParams(use_tc_tiling_on_sc=tc_tiling),
    )


_segsum_a = _make_segsum(DA, jnp.bfloat16, False)
_segsum_b = _make_segsum(DB, jnp.bfloat16, False)


@jax.jit
def kernel(features, edges, W1_l, b1, W1_r, W2_l, b2, W2_r):
    f32 = jnp.float32
    b1p = b1[None, :]
    b2p = b2[None, :]
    src_r = edges[0].reshape(NW, NCHUNK, CHUNK)
    dst_r = edges[1].reshape(NW, NCHUNK, CHUNK)
    zeros_a = jnp.zeros((NPAD, DA), jnp.bfloat16)
    zeros_b = jnp.zeros((NPAD, DB), jnp.bfloat16)

    y1a, y1b = pl.pallas_call(
        _k1_body,
        grid=(GRID,),
        in_specs=[pl.BlockSpec((BM, D_IN), lambda i: (i, 0)),
                  pl.BlockSpec((D_HID, D_IN), lambda i: (0, 0))],
        out_specs=[pl.BlockSpec((BM, DA), lambda i: (i, 0)),
                   pl.BlockSpec((BM, DB), lambda i: (i, 0))],
        out_shape=[jax.ShapeDtypeStruct((N, DA), jnp.bfloat16),
                   jax.ShapeDtypeStruct((N, DB), jnp.bfloat16)],
    )(features, W1_l)

    seg1a = _segsum_a(y1a, src_r, dst_r, zeros_a)
    seg1b = _segsum_b(y1b, src_r, dst_r, zeros_b)

    h2a, h2b = pl.pallas_call(
        _k3_body,
        grid=(GRID,),
        in_specs=[pl.BlockSpec((NC, BM, DA), lambda i: (0, i, 0)),
                  pl.BlockSpec((NC, BM, DB), lambda i: (0, i, 0)),
                  pl.BlockSpec((BM, D_IN), lambda i: (i, 0)),
                  pl.BlockSpec((D_HID, D_IN), lambda i: (0, 0)),
                  pl.BlockSpec((1, D_HID), lambda i: (0, 0))],
        out_specs=[pl.BlockSpec((BM, DA), lambda i: (i, 0)),
                   pl.BlockSpec((BM, DB), lambda i: (i, 0))],
        out_shape=[jax.ShapeDtypeStruct((N, DA), jnp.bfloat16),
                   jax.ShapeDtypeStruct((N, DB), jnp.bfloat16)],
    )(seg1a, seg1b, features, W1_r, b1p)

    seg2a = _segsum_a(h2a, src_r, dst_r, zeros_a)
    seg2b = _segsum_b(h2b, src_r, dst_r, zeros_b)

    out = pl.pallas_call(
        _k5_body,
        grid=(GRID,),
        in_specs=[pl.BlockSpec((NC, BM, DA), lambda i: (0, i, 0)),
                  pl.BlockSpec((NC, BM, DB), lambda i: (0, i, 0)),
                  pl.BlockSpec((BM, DA), lambda i: (i, 0)),
                  pl.BlockSpec((BM, DB), lambda i: (i, 0)),
                  pl.BlockSpec((D_IN, D_HID), lambda i: (0, 0)),
                  pl.BlockSpec((D_IN, D_HID), lambda i: (0, 0)),
                  pl.BlockSpec((1, D_IN), lambda i: (0, 0))],
        out_specs=pl.BlockSpec((BM, D_IN), lambda i: (i, 0)),
        out_shape=jax.ShapeDtypeStruct((N, D_IN), f32),
    )(seg2a, seg2b, h2a, h2b, W2_l, W2_r, b2p)
    return out
